# baseline (speedup 1.0000x reference)
import functools

import jax
import jax.numpy as jnp
from jax import lax
from jax.experimental import pallas as pl
from jax.experimental.pallas import tpu as pltpu
from jax.experimental.pallas import tpu_sc as plsc

N = 10000
E = 320000
D_IN = 128
D_OUT = 16

NC = 2
NS = 16
CH = 128

KD = 160
KE = 80
TOT = NS * KD
E_PAD = TOT * CH

NP = 10240
RPT = NP // NS

NB = 8
NGRP = KE // NB
WIN = 28

_mesh = plsc.VectorSubcoreMesh(core_axis_name="c", subcore_axis_name="s")
_sc_params = pltpu.CompilerParams(use_tc_tiling_on_sc=False)


@functools.partial(
    pl.kernel,
    mesh=_mesh,
    out_type=[
        jax.ShapeDtypeStruct((NP,), jnp.float32),
        jax.ShapeDtypeStruct((NC, NP, D_OUT), jnp.float32),
    ],
    scratch_types=(
        [
            pltpu.VMEM((KD, CH), jnp.int32),
            pltpu.VMEM((KE, CH), jnp.int32),
            pltpu.VMEM((CH,), jnp.float32),
            pltpu.VMEM((RPT,), jnp.float32),
            pltpu.VMEM((RPT, D_OUT), jnp.float32),
            pltpu.VMEM((NB, CH, D_OUT), jnp.float32),
            pltpu.VMEM_SHARED((NP,), jnp.float32),
            pltpu.VMEM_SHARED((NP, D_OUT), jnp.float32),
            pltpu.VMEM_SHARED((NP, D_OUT), jnp.float32),
        ]
        + [pltpu.SemaphoreType.DMA] * (2 + 2 * NB)
    ),
    compiler_params=_sc_params,
)
def _sc_mega(xw_hbm, row_hbm, col_hbm, ones_hbm, zer1_hbm, zer16_hbm,
             deg_hbm, acc_hbm,
             cidx_v, ridx_v, one_v, dbuf, xbuf, rows_v,
             deg_sh, y_sh, acc_sh, *sems):
    dsem = sems[0]
    xsem = sems[1]
    gsem = sems[2:2 + NB]
    ssem = sems[2 + NB:]
    c = lax.axis_index("c")
    s = lax.axis_index("s")

    pltpu.sync_copy(ones_hbm, one_v)
    pltpu.sync_copy(col_hbm.at[pl.ds(s * KD, KD)], cidx_v)
    pltpu.sync_copy(row_hbm.at[pl.ds(s * KD + c * KE, KE)], ridx_v)
    pltpu.sync_copy(zer1_hbm, deg_sh.at[pl.ds(s * RPT, RPT)])
    pltpu.sync_copy(zer16_hbm, acc_sh.at[pl.ds(s * RPT, RPT)])
    pltpu.async_copy(xw_hbm.at[pl.ds(s * RPT, RPT)], xbuf, xsem)
    plsc.subcore_barrier()

    def dfire(j):
        pltpu.async_copy(one_v, deg_sh.at[cidx_v.at[j]], dsem, add=True)

    def dwait():
        pltpu.make_async_copy(one_v, deg_sh.at[cidx_v.at[0]], dsem).wait()

    def dprol(j, carry):
        dfire(j)
        return carry

    lax.fori_loop(0, WIN, dprol, 0)

    def dsteady(j, carry):
        dwait()
        dfire(j + WIN)
        return carry

    lax.fori_loop(0, KD - WIN, dsteady, 0)

    def ddrain(j, carry):
        dwait()
        return carry

    lax.fori_loop(0, WIN, ddrain, 0)
    plsc.subcore_barrier()

    pltpu.sync_copy(deg_sh.at[pl.ds(s * RPT, RPT)], dbuf)
    @pl.when(c == 0)
    def _():
        pltpu.sync_copy(dbuf, deg_hbm.at[pl.ds(s * RPT, RPT)])
    pltpu.make_async_copy(xw_hbm.at[pl.ds(s * RPT, RPT)], xbuf, xsem).wait()

    def yrow(i, carry):
        d = dbuf[pl.ds(i * 16, 16)] + 1.0
        ib = lax.bitcast_convert_type(d, jnp.int32)
        ib = jnp.int32(0x5F3759DF) - lax.shift_right_arithmetic(ib, 1)
        r = lax.bitcast_convert_type(ib, jnp.float32)
        r = r * (1.5 - 0.5 * d * r * r)
        r = r * (1.5 - 0.5 * d * r * r)
        r = r * (1.5 - 0.5 * d * r * r)
        for t in range(16):
            rt = jax.lax.gather(
                r,
                jnp.full((16, 1), t, jnp.int32),
                jax.lax.GatherDimensionNumbers(
                    offset_dims=(), collapsed_slice_dims=(0,),
                    start_index_map=(0,)),
                (1,),
                mode=jax.lax.GatherScatterMode.PROMISE_IN_BOUNDS,
            )
            row = i * 16 + t
            xbuf[row, :] = xbuf[row, :] * rt
        return carry

    lax.fori_loop(0, RPT // 16, yrow, 0)
    pltpu.sync_copy(xbuf, y_sh.at[pl.ds(s * RPT, RPT)])
    plsc.subcore_barrier()

    ebase = c * KE

    def gsrc(j):
        return y_sh.at[ridx_v.at[j]]

    def sdst(j):
        return acc_sh.at[cidx_v.at[ebase + j]]

    for b in range(NB):
        pltpu.async_copy(gsrc(b), rows_v.at[b], gsem[b])

    def group(jo, carry):
        for b in range(NB):
            j = jo * NB + b
            pltpu.make_async_copy(gsrc(j), rows_v.at[b], gsem[b]).wait()
            pltpu.async_copy(rows_v.at[b], sdst(j), ssem[b], add=True)
            pltpu.make_async_copy(rows_v.at[b], sdst(j), ssem[b]).wait()
            pltpu.async_copy(gsrc(j + NB), rows_v.at[b], gsem[b])
        return carry

    lax.fori_loop(0, NGRP - 1, group, 0)

    for b in range(NB):
        j = (NGRP - 1) * NB + b
        pltpu.make_async_copy(gsrc(j), rows_v.at[b], gsem[b]).wait()
        pltpu.async_copy(rows_v.at[b], sdst(j), ssem[b], add=True)
    for b in range(NB):
        pltpu.make_async_copy(rows_v.at[b], sdst(0), ssem[b]).wait()

    plsc.subcore_barrier()
    pltpu.sync_copy(
        acc_sh.at[pl.ds(s * RPT, RPT)], acc_hbm.at[c, pl.ds(s * RPT, RPT)]
    )


def _xw_body(x_ref, w_ref, y_ref):
    y_ref[pl.ds(0, N), :] = jnp.dot(
        x_ref[...], w_ref[...], preferred_element_type=jnp.float32)
    y_ref[pl.ds(N, NP - N), :] = jnp.zeros((NP - N, D_OUT), jnp.float32)


def _xw_call(x, W):
    return pl.pallas_call(
        _xw_body,
        out_shape=jax.ShapeDtypeStruct((NP, D_OUT), jnp.float32),
    )(x, W)


def _fin_body(acc_ref, deg_ref, xw_ref, b_ref, out_ref):
    dis = lax.rsqrt(deg_ref[...] + 1.0)
    y = xw_ref[:N, :] * dis
    t = (acc_ref[0, :N, :] + acc_ref[1, :N, :] + y) * dis + b_ref[...]
    m = jnp.max(t, axis=1, keepdims=True)
    ls = jnp.log(jnp.sum(jnp.exp(t - m), axis=1, keepdims=True))
    out_ref[...] = t - m - ls


def _fin_call(acc_parts, deg, xw, b2d):
    return pl.pallas_call(
        _fin_body,
        out_shape=jax.ShapeDtypeStruct((N, D_OUT), jnp.float32),
    )(acc_parts, deg[:N].reshape(N, 1), xw, b2d)


@jax.jit
def kernel(x, edge_index, W, b):
    row = edge_index[0]
    col = edge_index[1]
    pad = E_PAD - E
    rowp = jnp.concatenate(
        [row, jnp.zeros((pad,), jnp.int32)]).reshape(TOT, CH)
    colp = jnp.concatenate(
        [col, jnp.full((pad,), N, jnp.int32)]).reshape(TOT, CH)

    ones_v = jnp.ones((CH,), jnp.float32)
    zer1 = jnp.zeros((RPT,), jnp.float32)
    zer16 = jnp.zeros((RPT, D_OUT), jnp.float32)

    xw = _xw_call(x, W)
    deg, acc_parts = _sc_mega(xw, rowp, colp, ones_v, zer1, zer16)
    return _fin_call(acc_parts, deg, xw, b.reshape(1, D_OUT))
